# baseline (speedup 1.0000x reference)
import functools

import jax
import jax.numpy as jnp
from jax import lax
from jax.experimental import pallas as pl
from jax.experimental.pallas import tpu as pltpu
from jax.experimental.pallas import tpu_sc as plsc

KB = 3
KQ = 5
_NEG = -jnp.inf
_IMAX = 2**31 - 1


def _ctx_body(enc_ref, out_ref, acc_s, *, nc, inv):
    j = pl.program_id(1)

    @pl.when(j == 0)
    def _():
        acc_s[...] = jnp.zeros_like(acc_s)

    acc_s[...] += jnp.sum(enc_ref[...], axis=1)

    @pl.when(j == nc - 1)
    def _():
        out_ref[...] = acc_s[...] * inv


def _pooled_ctx(enc):
    b, seq, d = enc.shape
    bb, ch = 8, 256
    nc = seq // ch
    return pl.pallas_call(
        functools.partial(_ctx_body, nc=nc, inv=1.0 / seq),
        grid=(b // bb, nc),
        in_specs=[pl.BlockSpec((bb, ch, d), lambda i, j: (i, j, 0))],
        out_specs=pl.BlockSpec((bb, d), lambda i, j: (i, 0)),
        out_shape=jax.ShapeDtypeStruct((b, d), jnp.float32),
        scratch_shapes=[pltpu.VMEM((bb, d), jnp.float32)],
    )(enc)


def _group3(x, rmod):
    xm1 = jnp.roll(x, 1, axis=0)
    xm2 = jnp.roll(x, 2, axis=0)
    xp1 = jnp.roll(x, -1, axis=0)
    xp2 = jnp.roll(x, -2, axis=0)
    a0 = jnp.where(rmod == 0, x, jnp.where(rmod == 1, xm1, xm2))
    a1 = jnp.where(rmod == 0, xp1, jnp.where(rmod == 1, x, xm1))
    a2 = jnp.where(rmod == 0, xp2, jnp.where(rmod == 1, xp1, x))
    return a0, a1, a2


def _step_body(emb_ref, ctx_ref, wh_ref, wout_ref, cum_ref, beams_ref,
               ncum_ref, nbeams_ref, tok_ref,
               h_s, m_s, s_s, tv_s, ti_s, *, nt, vt, vocab, t):
    i = pl.program_id(0)

    @pl.when(i == 0)
    def _():
        x = emb_ref[...] + ctx_ref[...]
        h_s[...] = jnp.tanh(jnp.dot(x, wh_ref[...],
                                    preferred_element_type=jnp.float32))
        m_s[...] = jnp.full_like(m_s, _NEG)
        s_s[...] = jnp.zeros_like(s_s)
        tv_s[...] = jnp.full_like(tv_s, _NEG)
        ti_s[...] = jnp.zeros_like(ti_s)

    logits = jnp.dot(h_s[...], wout_ref[...],
                     preferred_element_type=jnp.float32)
    colf = lax.broadcasted_iota(
        jnp.int32, logits.shape, 1).astype(jnp.float32)
    if nt * vt != vocab:
        limit = (vocab - i * vt).astype(jnp.float32)
        masked = jnp.where(colf < limit, logits, _NEG)
    else:
        masked = logits

    tile_m = jnp.max(masked, axis=1, keepdims=True)
    new_m = jnp.maximum(m_s[...], tile_m)
    e = jnp.exp(masked - new_m)
    s_s[...] = (s_s[...] * jnp.exp(m_s[...] - new_m)
                + jnp.sum(e, axis=1, keepdims=True))
    m_s[...] = new_m

    tv = tv_s[...]
    ti = ti_s[...]
    vs = [tv[:, q:q + 1] for q in range(KQ)]
    ix = [ti[:, q:q + 1] for q in range(KQ)]
    cur = masked
    cm = tile_m
    for r in range(KQ):
        eq = cur == cm
        cidxf = jnp.min(jnp.where(eq, colf, float(vt)),
                        axis=1, keepdims=True)
        cidx = i * vt + cidxf.astype(jnp.int32)
        g = [cm > v for v in vs]
        nvs, nix = [jnp.where(g[0], cm, vs[0])], [jnp.where(g[0], cidx, ix[0])]
        for q in range(1, KQ):
            nvs.append(jnp.where(g[q - 1], vs[q - 1],
                                 jnp.where(g[q], cm, vs[q])))
            nix.append(jnp.where(g[q - 1], ix[q - 1],
                                 jnp.where(g[q], cidx, ix[q])))
        vs, ix = nvs, nix
        if r < KQ - 1:
            cur = jnp.where(colf == cidxf, _NEG, cur)
            cm = jnp.max(cur, axis=1, keepdims=True)
    tv_s[...] = jnp.concatenate(vs, axis=1)
    ti_s[...] = jnp.concatenate(ix, axis=1)

    @pl.when(i == nt - 1)
    def _():
        logs = jnp.log(s_s[...])
        cum = cum_ref[...]
        cand = ((tv_s[...] - m_s[...]) - logs) + cum
        rows = lax.broadcasted_iota(jnp.int32, cand.shape, 0)
        rmod = rows - (rows // KB) * KB
        rmod1 = rmod[:, 0:1]
        c0, c1, c2 = _group3(cand, rmod)
        allc = jnp.concatenate([c0, c1, c2], axis=1)
        ti = ti_s[...]
        t0, t1, t2 = _group3(ti, rmod)
        keys = jnp.concatenate(
            [t0, t1 + vocab, t2 + 2 * vocab], axis=1)

        beams = beams_ref[...]
        rmodb = rmod1 + jnp.zeros_like(beams)
        b0, b1, b2 = _group3(beams, rmodb)
        pos = lax.broadcasted_iota(jnp.int32, beams.shape, 1)

        cur = allc
        vs, ts, ps = [], [], []
        for _ in range(KB):
            m = jnp.max(cur, axis=1, keepdims=True)
            eq = cur == m
            fk = jnp.min(jnp.where(eq, keys, _IMAX),
                         axis=1, keepdims=True)
            sel = keys == fk
            vs.append(m)
            prev = fk // vocab
            ps.append(prev)
            ts.append(fk - prev * vocab)
            cur = jnp.where(sel, _NEG, cur)

        pick = lambda xs: jnp.where(
            rmod1 == 0, xs[0], jnp.where(rmod1 == 1, xs[1], xs[2]))
        ncum_ref[...] = pick(vs)
        tok = pick(ts)
        tok_ref[...] = tok
        prev = pick(ps)
        nb = jnp.where(prev == 0, b0, jnp.where(prev == 1, b1, b2))
        nbeams_ref[...] = jnp.where(pos == t, tok, nb)


def _fused_step(emb, ctx_rows, W_h, W_out, cum_row, beams, t, vt=2048):
    r, d = emb.shape
    vocab = W_out.shape[1]
    tdec = beams.shape[1]
    nt = pl.cdiv(vocab, vt)
    full = lambda i: (0, 0)
    return pl.pallas_call(
        functools.partial(_step_body, nt=nt, vt=vt, vocab=vocab, t=t),
        grid=(nt,),
        in_specs=[
            pl.BlockSpec((r, d), full),
            pl.BlockSpec((r, d), full),
            pl.BlockSpec((d, d), full),
            pl.BlockSpec((d, vt), lambda i: (0, i)),
            pl.BlockSpec((r, 1), full),
            pl.BlockSpec((r, tdec), full),
        ],
        out_specs=[
            pl.BlockSpec((r, 1), full),
            pl.BlockSpec((r, tdec), full),
            pl.BlockSpec((r, 1), full),
        ],
        out_shape=[
            jax.ShapeDtypeStruct((r, 1), jnp.float32),
            jax.ShapeDtypeStruct((r, tdec), jnp.int32),
            jax.ShapeDtypeStruct((r, 1), jnp.int32),
        ],
        scratch_shapes=[
            pltpu.VMEM((r, d), jnp.float32),
            pltpu.VMEM((r, 1), jnp.float32),
            pltpu.VMEM((r, 1), jnp.float32),
            pltpu.VMEM((r, KQ), jnp.float32),
            pltpu.VMEM((r, KQ), jnp.int32),
        ],
    )(emb, ctx_rows, W_h, W_out, cum_row, beams)


def _sc_gather(tok, table):
    r = tok.shape[0]
    d = table.shape[1]
    nw = r // 8
    mesh = plsc.VectorSubcoreMesh(core_axis_name="c", subcore_axis_name="s")

    @functools.partial(
        pl.kernel,
        out_type=jax.ShapeDtypeStruct((r, d), jnp.float32),
        mesh=mesh,
        scratch_types=[pltpu.VMEM((8,), jnp.int32),
                       pltpu.VMEM((8, d), jnp.float32),
                       pltpu.SemaphoreType.DMA],
    )
    def gk(tok_hbm, table_hbm, out_hbm, idx_v, rows_v, sem):
        wid = lax.axis_index("s") * 2 + lax.axis_index("c")

        @pl.when(wid < nw)
        def _():
            base = wid * 8
            pltpu.sync_copy(tok_hbm.at[pl.ds(base, 8)], idx_v)
            pltpu.async_copy(table_hbm.at[idx_v], rows_v, sem).wait()
            pltpu.sync_copy(rows_v, out_hbm.at[pl.ds(base, 8)])

    return gk(tok, table)


def kernel(input_var, encoder_outputs, k, W_emb, W_h, W_out):
    bsz = encoder_outputs.shape[0]
    tdec = 8
    r = bsz * KB

    ctx = _pooled_ctx(encoder_outputs)
    ctx_k = jnp.repeat(ctx, KB, axis=0)
    vt = 8192

    cum_row = jnp.where(jnp.arange(r) % KB == 0,
                        0.0, -jnp.inf)[:, None].astype(jnp.float32)
    beams = jnp.zeros((r, tdec), jnp.int32)
    last = jnp.repeat(input_var[:, 0], KB)

    for t in range(tdec):
        emb = _sc_gather(last, W_emb)
        cum_row, beams, tok = _fused_step(
            emb, ctx_k, W_h, W_out, cum_row, beams, t, vt=vt)
        last = tok.reshape(r)

    hyp = beams.reshape(bsz, KB, tdec)[:, 0, :]
    return hyp, cum_row.reshape(bsz, KB)
